# baseline (speedup 1.0000x reference)
import functools

import jax
import jax.numpy as jnp
from jax import lax
from jax.experimental import pallas as pl
from jax.experimental.pallas import tpu as pltpu
from jax.experimental.pallas import tpu_sc as plsc

B = 16384
D = 64
NUM_PTM = 26
PER = 128
K = NUM_PTM * PER
COMMIT = 0.25
TEMP = 0.07
SAMPLED = int(0.1 * PER) * NUM_PTM
SPAD = 384

TB = 512
DPAD = 128
N_SC_CORES = 2
N_SC_SUBCORES = 16
NW = N_SC_CORES * N_SC_SUBCORES
BPW = B // NW


def _tc_body(x_ref, q_ref, tab_ref, e2_ref, sidx_ref,
             idx_ref, tablen_ref, lsum_ref, ul_ref):
    step = pl.program_id(0)

    xr = x_ref[...]
    norm = jnp.sqrt(jnp.sum(xr * xr, axis=1, keepdims=True))
    xn = xr / jnp.maximum(norm, 1e-12)
    c = jnp.sum(xn * xn, axis=1, keepdims=True)

    s = lax.dot_general(xn, tab_ref[...], (((1,), (1,)), ((), ())),
                        preferred_element_type=jnp.float32)
    d2 = (c + e2_ref[...]) - 2.0 * s

    d3 = d2.reshape(TB, NUM_PTM, PER)
    m26 = jnp.min(d3, axis=2)
    lane7 = lax.broadcasted_iota(jnp.int32, (TB, NUM_PTM, PER), 2)
    w26 = jnp.min(jnp.where(d3 == m26[:, :, None], lane7, PER), axis=2)
    wsel = lax.broadcasted_iota(jnp.int32, (TB, NUM_PTM), 1) == q_ref[...]
    m = jnp.sum(jnp.where(wsel, m26, 0.0), axis=1, keepdims=True)
    widx = jnp.sum(jnp.where(wsel, w26, 0), axis=1, keepdims=True)
    idx_ref[...] = q_ref[...] * PER + widx

    @pl.when(step == 0)
    def _():
        lsum_ref[...] = jnp.zeros_like(lsum_ref)
        tab = tab_ref[...]
        tn = tab / jnp.maximum(
            jnp.sqrt(jnp.sum(tab * tab, axis=1, keepdims=True)), 1e-12)
        tablen_ref[...] = jnp.concatenate(
            [tn, jnp.zeros((K, DPAD - D), jnp.float32)], axis=1)
        si = sidx_ref[...]
        onehot = (si.reshape(SPAD, 1)
                  == lax.broadcasted_iota(jnp.int32, (SPAD, K), 1))
        se = lax.dot_general(onehot.astype(jnp.float32), tn,
                             (((1,), (0,)), ((), ())),
                             preferred_element_type=jnp.float32,
                             precision=lax.Precision.HIGHEST)
        sim = lax.dot_general(se, se, (((1,), (1,)), ((), ())),
                              preferred_element_type=jnp.float32,
                              precision=lax.Precision.HIGHEST)
        valid = si.reshape(1, SPAD) >= 0
        eye = (lax.broadcasted_iota(jnp.int32, (SPAD, SPAD), 0)
               == lax.broadcasted_iota(jnp.int32, (SPAD, SPAD), 1))
        keep = valid & jnp.logical_not(eye)
        simm = jnp.where(keep, sim, -jnp.float32(jnp.inf))
        ex = jnp.exp(simm / TEMP)
        sum_exp = jnp.sum(ex, axis=1, keepdims=True)
        labels = jnp.where(si >= 0, si // PER, -1)
        pos = labels.reshape(SPAD, 1) == labels.reshape(1, SPAD)
        pos_sum = jnp.sum(jnp.where(pos, ex, 0.0), axis=1, keepdims=True)
        validc = si.reshape(SPAD, 1) >= 0
        ratio = jnp.where(validc, pos_sum / jnp.maximum(sum_exp, 1e-30), 1.0)
        ul = -jnp.sum(jnp.log(ratio)) / SAMPLED
        ul_ref[...] = jnp.full((1, 1), ul, dtype=jnp.float32)

    lsum_ref[...] += jnp.sum(m).reshape(1, 1)


def _tc_call(x, q2, table, e2, sidx, interpret=False):
    grid = B // TB
    return pl.pallas_call(
        _tc_body,
        grid=(grid,),
        in_specs=[
            pl.BlockSpec((TB, D), lambda i: (i, 0)),
            pl.BlockSpec((TB, 1), lambda i: (i, 0)),
            pl.BlockSpec((K, D), lambda i: (0, 0)),
            pl.BlockSpec((1, K), lambda i: (0, 0)),
            pl.BlockSpec((1, SPAD), lambda i: (0, 0)),
        ],
        out_specs=[
            pl.BlockSpec((TB, 1), lambda i: (i, 0)),
            pl.BlockSpec((K, DPAD), lambda i: (0, 0)),
            pl.BlockSpec((1, 1), lambda i: (0, 0)),
            pl.BlockSpec((1, 1), lambda i: (0, 0)),
        ],
        out_shape=[
            jax.ShapeDtypeStruct((B, 1), jnp.int32),
            jax.ShapeDtypeStruct((K, DPAD), jnp.float32),
            jax.ShapeDtypeStruct((1, 1), jnp.float32),
            jax.ShapeDtypeStruct((1, 1), jnp.float32),
        ],
        interpret=interpret,
    )(x, q2, table, e2, sidx)


def _sc_gather(tablen, idx):
    mesh = plsc.VectorSubcoreMesh(core_axis_name="c", subcore_axis_name="s")

    @functools.partial(
        pl.kernel,
        out_type=jax.ShapeDtypeStruct((B, DPAD), jnp.float32),
        mesh=mesh,
        scratch_types=[
            pltpu.VMEM((BPW,), jnp.int32),
            pltpu.VMEM((BPW, DPAD), jnp.float32),
            pltpu.SemaphoreType.DMA,
        ],
    )
    def gk(table_hbm, idx_hbm, out_hbm, idx_v, rows_v, sem):
        wid = lax.axis_index("s") * N_SC_CORES + lax.axis_index("c")
        base = wid * BPW
        pltpu.sync_copy(idx_hbm.at[pl.ds(base, BPW)], idx_v)
        pltpu.async_copy(table_hbm.at[idx_v], rows_v, sem).wait()
        pltpu.sync_copy(rows_v, out_hbm.at[pl.ds(base, BPW)])

    return gk(tablen, idx)


def _sampled_indices():
    perm = jax.random.permutation(jax.random.key(42), PER)[:int(0.1 * PER)]
    all_idx = jnp.arange(K).reshape(NUM_PTM, PER)
    si = all_idx[:, perm].reshape(-1).astype(jnp.int32)
    return jnp.concatenate(
        [si, jnp.full((SPAD - SAMPLED,), -1, jnp.int32)]).reshape(1, SPAD)


def kernel(x, Q, embeddings):
    e2 = jnp.sum(embeddings ** 2, axis=1)[None, :]
    q2 = Q.reshape(B, 1)
    sidx = _sampled_indices()
    idx2, tablen, lsum, ul = _tc_call(x, q2, embeddings, e2, sidx)
    idx = idx2.reshape(B)
    quantized = _sc_gather(tablen, idx)[:, :D]
    loss = lsum[0, 0] * ((1.0 + COMMIT) / (B * D))
    return (quantized, loss, ul[0, 0], idx)


# speedup vs baseline: 2.1691x; 2.1691x over previous
import functools

import jax
import jax.numpy as jnp
from jax import lax
from jax.experimental import pallas as pl
from jax.experimental.pallas import tpu as pltpu
from jax.experimental.pallas import tpu_sc as plsc

B = 16384
D = 64
NUM_PTM = 26
PER = 128
K = NUM_PTM * PER
COMMIT = 0.25
TEMP = 0.07
SAMPLED = int(0.1 * PER) * NUM_PTM
SPAD = 384

TB = 2048
DPAD = 128
N_SC_CORES = 2
N_SC_SUBCORES = 16
NW = N_SC_CORES * N_SC_SUBCORES
BPW = B // NW


def _tc_body(x_ref, q_ref, tab_ref, e2_ref, sidx_ref,
             idx_ref, tablen_ref, lsum_ref, ul_ref):
    step = pl.program_id(0)

    xr = x_ref[...]
    norm = jnp.sqrt(jnp.sum(xr * xr, axis=1, keepdims=True))
    xn = xr / jnp.maximum(norm, 1e-12)
    c = jnp.sum(xn * xn, axis=1, keepdims=True)

    s = lax.dot_general(xn, tab_ref[...], (((1,), (1,)), ((), ())),
                        preferred_element_type=jnp.float32)
    d2 = (c + e2_ref[...]) - 2.0 * s

    coltype = lax.broadcasted_iota(jnp.int32, (1, K), 1) // PER
    mask = coltype == q_ref[...]
    inf = jnp.float32(jnp.inf)
    d2m = jnp.where(mask, d2, inf)
    m = jnp.min(d2m, axis=1, keepdims=True)
    lane = lax.broadcasted_iota(jnp.int32, (TB, K), 1)
    idx = jnp.min(jnp.where(d2m == m, lane, K), axis=1)
    idx_ref[...] = idx[:, None]

    @pl.when(step == 0)
    def _():
        lsum_ref[...] = jnp.zeros_like(lsum_ref)
        tab = tab_ref[...]
        tn = tab / jnp.maximum(
            jnp.sqrt(jnp.sum(tab * tab, axis=1, keepdims=True)), 1e-12)
        tablen_ref[...] = jnp.concatenate(
            [tn, jnp.zeros((K, DPAD - D), jnp.float32)], axis=1)
        si = sidx_ref[...]
        onehot = (si.reshape(SPAD, 1)
                  == lax.broadcasted_iota(jnp.int32, (SPAD, K), 1))
        se = lax.dot_general(onehot.astype(jnp.float32), tn,
                             (((1,), (0,)), ((), ())),
                             preferred_element_type=jnp.float32,
                             precision=lax.Precision.HIGHEST)
        sim = lax.dot_general(se, se, (((1,), (1,)), ((), ())),
                              preferred_element_type=jnp.float32,
                              precision=lax.Precision.HIGHEST)
        valid = si.reshape(1, SPAD) >= 0
        eye = (lax.broadcasted_iota(jnp.int32, (SPAD, SPAD), 0)
               == lax.broadcasted_iota(jnp.int32, (SPAD, SPAD), 1))
        keep = valid & jnp.logical_not(eye)
        simm = jnp.where(keep, sim, -jnp.float32(jnp.inf))
        ex = jnp.exp(simm / TEMP)
        sum_exp = jnp.sum(ex, axis=1, keepdims=True)
        labels = jnp.where(si >= 0, si // PER, -1)
        pos = labels.reshape(SPAD, 1) == labels.reshape(1, SPAD)
        pos_sum = jnp.sum(jnp.where(pos, ex, 0.0), axis=1, keepdims=True)
        validc = si.reshape(SPAD, 1) >= 0
        ratio = jnp.where(validc, pos_sum / jnp.maximum(sum_exp, 1e-30), 1.0)
        ul = -jnp.sum(jnp.log(ratio)) / SAMPLED
        ul_ref[...] = jnp.full((1, 1), ul, dtype=jnp.float32)

    lsum_ref[...] += jnp.sum(m).reshape(1, 1)


def _tc_call(x, q2, table, e2, sidx, interpret=False):
    grid = B // TB
    return pl.pallas_call(
        _tc_body,
        grid=(grid,),
        in_specs=[
            pl.BlockSpec((TB, D), lambda i: (i, 0)),
            pl.BlockSpec((TB, 1), lambda i: (i, 0)),
            pl.BlockSpec((K, D), lambda i: (0, 0)),
            pl.BlockSpec((1, K), lambda i: (0, 0)),
            pl.BlockSpec((1, SPAD), lambda i: (0, 0)),
        ],
        out_specs=[
            pl.BlockSpec((TB, 1), lambda i: (i, 0)),
            pl.BlockSpec((K, DPAD), lambda i: (0, 0)),
            pl.BlockSpec((1, 1), lambda i: (0, 0)),
            pl.BlockSpec((1, 1), lambda i: (0, 0)),
        ],
        out_shape=[
            jax.ShapeDtypeStruct((B, 1), jnp.int32),
            jax.ShapeDtypeStruct((K, DPAD), jnp.float32),
            jax.ShapeDtypeStruct((1, 1), jnp.float32),
            jax.ShapeDtypeStruct((1, 1), jnp.float32),
        ],
        interpret=interpret,
    )(x, q2, table, e2, sidx)


def _sc_gather(tablen, idx):
    mesh = plsc.VectorSubcoreMesh(core_axis_name="c", subcore_axis_name="s")

    @functools.partial(
        pl.kernel,
        out_type=jax.ShapeDtypeStruct((B, DPAD), jnp.float32),
        mesh=mesh,
        scratch_types=[
            pltpu.VMEM((BPW,), jnp.int32),
            pltpu.VMEM((BPW, DPAD), jnp.float32),
            pltpu.SemaphoreType.DMA,
        ],
    )
    def gk(table_hbm, idx_hbm, out_hbm, idx_v, rows_v, sem):
        wid = lax.axis_index("s") * N_SC_CORES + lax.axis_index("c")
        base = wid * BPW
        pltpu.sync_copy(idx_hbm.at[pl.ds(base, BPW)], idx_v)
        pltpu.async_copy(table_hbm.at[idx_v], rows_v, sem).wait()
        pltpu.sync_copy(rows_v, out_hbm.at[pl.ds(base, BPW)])

    return gk(tablen, idx)


def _sampled_indices():
    perm = jax.random.permutation(jax.random.key(42), PER)[:int(0.1 * PER)]
    all_idx = jnp.arange(K).reshape(NUM_PTM, PER)
    si = all_idx[:, perm].reshape(-1).astype(jnp.int32)
    return jnp.concatenate(
        [si, jnp.full((SPAD - SAMPLED,), -1, jnp.int32)]).reshape(1, SPAD)


def kernel(x, Q, embeddings):
    e2 = jnp.sum(embeddings ** 2, axis=1)[None, :]
    q2 = Q.reshape(B, 1)
    sidx = _sampled_indices()
    idx2, tablen, lsum, ul = _tc_call(x, q2, embeddings, e2, sidx)
    idx = idx2.reshape(B)
    quantized = _sc_gather(tablen, idx)[:, :D]
    loss = lsum[0, 0] * ((1.0 + COMMIT) / (B * D))
    return (quantized, loss, ul[0, 0], idx)
